# baseline (speedup 1.0000x reference)
import functools

import jax
import jax.numpy as jnp
from jax import lax
from jax.experimental import pallas as pl
from jax.experimental.pallas import tpu as pltpu
from jax.experimental.pallas import tpu_sc as plsc

_N0, _N1 = 6000, 4000
_N = _N0 + _N1
_E = 320000
_D = 128
_R = 4
_NC, _NS = 2, 16
_NW = _NC * _NS
_CH = 128
_NCHUNKS = _E // _CH
_CITER = 80
_STAGE = _CITER // 2
_NCHP = _NW * _CITER
_CNTP = 40960
_CSLICE = _CNTP // _NS
_ROWS_A = 624
_ROWS_LAST = _N - 15 * _ROWS_A

_mesh = plsc.VectorSubcoreMesh(core_axis_name="c", subcore_axis_name="s")


@functools.partial(
    pl.kernel,
    out_type=jax.ShapeDtypeStruct((_NC, _CNTP), jnp.float32),
    mesh=_mesh,
    compiler_params=pltpu.CompilerParams(needs_layout_passes=False),
    scratch_types=[
        pltpu.VMEM((_CITER, _CH), jnp.int32),
        pltpu.VMEM((_CITER, _CH), jnp.int32),
        pltpu.VMEM((_CITER, _CH), jnp.int32),
        pltpu.VMEM((_CH,), jnp.float32),
        pltpu.VMEM((_CSLICE,), jnp.float32),
        pltpu.SemaphoreType.DMA,
        pltpu.VMEM_SHARED((_CNTP,), jnp.float32),
    ],
)
def _sc_counts(dst_hbm, et_hbm, out_hbm, dst_a, et_a, seg_a, one_v, zb_v,
               sem, acc_sh):
    c = lax.axis_index("c")
    s = lax.axis_index("s")
    wid = c * _NS + s
    zeros16 = jnp.zeros((16,), jnp.float32)
    for k in range(_CSLICE // 16):
        zb_v[pl.ds(k * 16, 16)] = zeros16
    for k in range(_CH // 16):
        one_v[pl.ds(k * 16, 16)] = jnp.full((16,), 1.0, jnp.float32)
    cb = wid * _CITER
    pltpu.sync_copy(dst_hbm.at[pl.ds(cb, _CITER)], dst_a)
    pltpu.sync_copy(et_hbm.at[pl.ds(cb, _CITER)], et_a)
    pltpu.sync_copy(zb_v, acc_sh.at[pl.ds(s * _CSLICE, _CSLICE)])
    plsc.subcore_barrier()

    def body(j, carry):
        for k in range(_CH // 16):
            sl = pl.ds(k * 16, 16)
            r16 = jnp.right_shift(et_a[j, sl], 1)
            seg_a[j, sl] = r16 * _N + dst_a[j, sl]
        pltpu.async_copy(one_v, acc_sh.at[seg_a.at[j]], sem, add=True)
        return carry

    lax.fori_loop(0, _CITER, body, 0)

    def drain(j, carry):
        pltpu.make_async_copy(one_v, acc_sh.at[seg_a.at[j]], sem).wait()
        return carry

    lax.fori_loop(0, _CITER, drain, 0)
    plsc.subcore_barrier()
    pltpu.sync_copy(acc_sh.at[pl.ds(s * _CSLICE, _CSLICE)],
                    out_hbm.at[c, pl.ds(s * _CSLICE, _CSLICE)])


@functools.partial(
    pl.kernel,
    out_type=(jax.ShapeDtypeStruct((_NCHP, _CH), jnp.float32),
              jax.ShapeDtypeStruct((_NCHP, _CH), jnp.int32)),
    mesh=_mesh,
    compiler_params=pltpu.CompilerParams(needs_layout_passes=False),
    scratch_types=[
        pltpu.VMEM((_CITER, _CH), jnp.int32),
        pltpu.VMEM((_CITER, _CH), jnp.int32),
        pltpu.VMEM((_CITER, _CH), jnp.int32),
        pltpu.VMEM((_CITER, _CH), jnp.float32),
        pltpu.VMEM((_CITER, _CH), jnp.int32),
        pltpu.VMEM((_CNTP,), jnp.float32),
    ],
)
def _sc_edge_prep(src_hbm, dst_hbm, et_hbm, inv_hbm,
                  w_hbm, gidx_hbm, src_a, dst_a, et_a, w_a, g_a, inv_v):
    c = lax.axis_index("c")
    s = lax.axis_index("s")
    wid = c * _NS + s
    cb = wid * _CITER
    pltpu.sync_copy(inv_hbm, inv_v)
    pltpu.sync_copy(src_hbm.at[pl.ds(cb, _CITER)], src_a)
    pltpu.sync_copy(dst_hbm.at[pl.ds(cb, _CITER)], dst_a)
    pltpu.sync_copy(et_hbm.at[pl.ds(cb, _CITER)], et_a)

    def body(j, carry):
        for k in range(_CH // 16):
            sl = pl.ds(k * 16, 16)
            r16 = jnp.right_shift(et_a[j, sl], 1)
            seg16 = r16 * _N + dst_a[j, sl]
            w_a[j, sl] = plsc.load_gather(inv_v, [seg16])
            g_a[j, sl] = (r16 & 3) * _N + src_a[j, sl]
        return carry

    lax.fori_loop(0, _CITER, body, 0)
    pltpu.sync_copy(w_a, w_hbm.at[pl.ds(cb, _CITER)])
    pltpu.sync_copy(g_a, gidx_hbm.at[pl.ds(cb, _CITER)])


@functools.partial(
    pl.kernel,
    out_type=jax.ShapeDtypeStruct((_NC, _N, _D), jnp.float32),
    mesh=_mesh,
    compiler_params=pltpu.CompilerParams(needs_layout_passes=False),
    scratch_types=[
        pltpu.VMEM((_STAGE, _CH), jnp.int32),
        pltpu.VMEM((_STAGE, _CH), jnp.int32),
        pltpu.VMEM((_STAGE, _CH), jnp.float32),
        pltpu.VMEM((_CH, _D), jnp.float32),
        pltpu.VMEM((_CH, _D), jnp.float32),
        pltpu.SemaphoreType.DMA,
        pltpu.SemaphoreType.DMA,
        pltpu.VMEM_SHARED((_N, _D), jnp.float32),
    ],
)
def _sc_accumulate(z_hbm, gidx_hbm, dst_hbm, w_hbm, out_hbm,
                   gi_a, dst_a, w_a, rows0, rows1, sem0, sem1, acc_sh):
    c = lax.axis_index("c")
    s = lax.axis_index("s")
    wid = c * _NS + s
    zeros16 = jnp.zeros((16,), jnp.float32)

    def zrow(i, carry):
        for k in range(_D // 16):
            rows0[i, pl.ds(k * 16, 16)] = zeros16
        return carry

    lax.fori_loop(0, _CH, zrow, 0)
    base = s * _ROWS_A
    for q in range(4):
        pltpu.sync_copy(rows0, acc_sh.at[pl.ds(base + q * _CH, _CH)])

    @pl.when(s < _NS - 1)
    def _():
        pltpu.sync_copy(rows0.at[pl.ds(0, _ROWS_A - 4 * _CH)],
                        acc_sh.at[pl.ds(base + 4 * _CH, _ROWS_A - 4 * _CH)])

    @pl.when(s == _NS - 1)
    def _():
        pltpu.sync_copy(rows0, acc_sh.at[pl.ds(15 * _ROWS_A + 4 * _CH, _CH)])

    plsc.subcore_barrier()

    def scale(rows_v, j):
        jsp = jnp.full((16,), j, jnp.int32)

        def srow(i8, carry2):
            i = i8 * 8
            wsps = [
                plsc.load_gather(w_a, [jsp, jnp.full((16,), i + u, jnp.int32)])
                for u in range(8)
            ]
            for u in range(8):
                for k in range(_D // 16):
                    sl = pl.ds(k * 16, 16)
                    rows_v[i + u, sl] = rows_v[i + u, sl] * wsps[u]
            return carry2

        lax.fori_loop(0, _CH // 8, srow, 0)

    cb = wid * _CITER
    for half in range(2):
        sb = cb + half * _STAGE
        pltpu.sync_copy(gidx_hbm.at[pl.ds(sb, _STAGE)], gi_a)
        pltpu.sync_copy(dst_hbm.at[pl.ds(sb, _STAGE)], dst_a)
        pltpu.sync_copy(w_hbm.at[pl.ds(sb, _STAGE)], w_a)
        pltpu.async_copy(z_hbm.at[gi_a.at[0]], rows0, sem0)

        def body(i, carry):
            j0 = i * 2
            pltpu.async_copy(z_hbm.at[gi_a.at[j0 + 1]], rows1, sem1)
            pltpu.make_async_copy(z_hbm.at[gi_a.at[j0]], rows0, sem0).wait()
            scale(rows0, j0)
            pltpu.sync_copy(rows0, acc_sh.at[dst_a.at[j0]], add=True)

            @pl.when(j0 + 2 < _STAGE)
            def _():
                pltpu.async_copy(z_hbm.at[gi_a.at[j0 + 2]], rows0, sem0)

            pltpu.make_async_copy(z_hbm.at[gi_a.at[j0 + 1]], rows1, sem1).wait()
            scale(rows1, j0 + 1)
            pltpu.sync_copy(rows1, acc_sh.at[dst_a.at[j0 + 1]], add=True)
            return carry

        lax.fori_loop(0, _STAGE // 2, body, 0)
    plsc.subcore_barrier()

    @pl.when(s < _NS - 1)
    def _():
        pltpu.sync_copy(acc_sh.at[pl.ds(s * _ROWS_A, _ROWS_A)],
                        out_hbm.at[c, pl.ds(s * _ROWS_A, _ROWS_A)])

    @pl.when(s == _NS - 1)
    def _():
        pltpu.sync_copy(acc_sh.at[pl.ds(15 * _ROWS_A, _ROWS_LAST)],
                        out_hbm.at[c, pl.ds(15 * _ROWS_A, _ROWS_LAST)])


def _proj_body(x_ref, w_ref, b_ref, o_ref):
    o_ref[...] = jnp.dot(x_ref[...], w_ref[...],
                         preferred_element_type=jnp.float32) + b_ref[...]


def _tc_project(x, w, b, block):
    m, k = x.shape
    return pl.pallas_call(
        _proj_body,
        grid=(m // block,),
        in_specs=[
            pl.BlockSpec((block, k), lambda i: (i, 0)),
            pl.BlockSpec((k, _D), lambda i: (0, 0)),
            pl.BlockSpec((1, _D), lambda i: (0, 0)),
        ],
        out_specs=pl.BlockSpec((block, _D), lambda i: (i, 0)),
        out_shape=jax.ShapeDtypeStruct((m, _D), jnp.float32),
    )(x, w, b.reshape(1, _D))


def _inv_body(c_ref, o_ref):
    inv = 1.0 / jnp.maximum(c_ref[0] + c_ref[1], 1.0)
    rows = c_ref.shape[1]
    flat = (lax.broadcasted_iota(jnp.int32, (rows, _D), 0) * _D
            + lax.broadcasted_iota(jnp.int32, (rows, _D), 1))
    o_ref[...] = jnp.where(flat < _R * _N, inv, 0.0)


def _tc_inv_counts(cnt2):
    c3 = cnt2.reshape(_NC, _CNTP // _D, _D)
    out = pl.pallas_call(
        _inv_body,
        out_shape=jax.ShapeDtypeStruct((_CNTP // _D, _D), jnp.float32),
    )(c3)
    return out.reshape(_CNTP)


def _z_body(x_ref, comp_ref, bases_ref, o_ref):
    for r in range(_R):
        wt = comp_ref[r, 0] * bases_ref[0]
        for b in range(1, bases_ref.shape[0]):
            wt = wt + comp_ref[r, b] * bases_ref[b]
        o_ref[r] = jnp.dot(x_ref[...], wt, preferred_element_type=jnp.float32)


def _tc_ztable(x, comp, bases, block):
    nb = bases.shape[0]
    return pl.pallas_call(
        _z_body,
        grid=(_N // block,),
        in_specs=[
            pl.BlockSpec((block, _D), lambda i: (i, 0)),
            pl.BlockSpec(memory_space=pltpu.SMEM),
            pl.BlockSpec((nb, _D, _D), lambda i: (0, 0, 0)),
        ],
        out_specs=pl.BlockSpec((_R, block, _D), lambda i: (0, i, 0)),
        out_shape=jax.ShapeDtypeStruct((_R, _N, _D), jnp.float32),
    )(x, comp, bases)


def _combine_body(acc_ref, x_ref, root_ref, b_ref, g_ref, bb_ref, o_ref, *, act):
    h = (acc_ref[0] + acc_ref[1] + b_ref[...]
         + jnp.dot(x_ref[...], root_ref[...], preferred_element_type=jnp.float32))
    if act == "relu":
        h = jnp.maximum(h, 0.0)
    elif act == "ln":
        mu = jnp.mean(h, axis=-1, keepdims=True)
        xc = h - mu
        var = jnp.mean(xc * xc, axis=-1, keepdims=True)
        h = xc * lax.rsqrt(var + 1e-5) * g_ref[...] + bb_ref[...]
    o_ref[...] = h


def _tc_combine(acc2, x, root, bias, g, bb, act, block):
    body = functools.partial(_combine_body, act=act)
    return pl.pallas_call(
        body,
        grid=(_N // block,),
        in_specs=[
            pl.BlockSpec((_NC, block, _D), lambda i: (0, i, 0)),
            pl.BlockSpec((block, _D), lambda i: (i, 0)),
            pl.BlockSpec((_D, _D), lambda i: (0, 0)),
            pl.BlockSpec((1, _D), lambda i: (0, 0)),
            pl.BlockSpec((1, _D), lambda i: (0, 0)),
            pl.BlockSpec((1, _D), lambda i: (0, 0)),
        ],
        out_specs=pl.BlockSpec((block, _D), lambda i: (i, 0)),
        out_shape=jax.ShapeDtypeStruct((_N, _D), jnp.float32),
    )(acc2, x, root, bias.reshape(1, _D), g.reshape(1, _D), bb.reshape(1, _D))


def kernel(x0, x1, edge_index, edge_type, lin0_w, lin0_b, lin1_w, lin1_b,
           comp1, bases1, root1, bias1, comp2, bases2, root2, bias2,
           comp3, bases3, root3, bias3, ln_g, ln_b):
    block = 1000
    pad = 384 - x1.shape[1]
    x1p = jnp.pad(x1, ((0, 0), (0, pad)))
    w1p = jnp.pad(lin1_w, ((0, pad), (0, 0)))
    xa = _tc_project(x0, lin0_w, lin0_b, 1000)
    xb = _tc_project(x1p, w1p, lin1_b, 1000)
    x = jnp.concatenate([xa, xb], axis=0)

    npad = _NCHP * _CH - _E
    pad_idx = jnp.arange(npad, dtype=jnp.int32)
    srcp = jnp.concatenate([edge_index[0], pad_idx % _N]).reshape(_NCHP, _CH)
    dstp = jnp.concatenate([edge_index[1], pad_idx % 960]).reshape(_NCHP, _CH)
    etp = jnp.concatenate(
        [edge_type, jnp.full((npad,), 8, jnp.int32)]).reshape(_NCHP, _CH)
    cnt2 = _sc_counts(dstp, etp)
    inv = _tc_inv_counts(cnt2)
    w_e, gidx = _sc_edge_prep(srcp, dstp, etp, inv)

    layers = ((comp1, bases1, root1, bias1, "relu"),
              (comp2, bases2, root2, bias2, "relu"),
              (comp3, bases3, root3, bias3, "ln"))
    for comp, bases, root, bias, act in layers:
        z = _tc_ztable(x, comp, bases, block).reshape(_R * _N, _D)
        acc2 = _sc_accumulate(z, gidx, dstp, w_e)
        x = _tc_combine(acc2, x, root, bias, ln_g, ln_b, act, block)
    return x


# speedup vs baseline: 1.2971x; 1.2971x over previous
import functools

import jax
import jax.numpy as jnp
from jax import lax
from jax.experimental import pallas as pl
from jax.experimental.pallas import tpu as pltpu
from jax.experimental.pallas import tpu_sc as plsc

_N0, _N1 = 6000, 4000
_N = _N0 + _N1
_E = 320000
_D = 128
_R = 4
_NC, _NS = 2, 16
_NW = _NC * _NS
_CH = 128
_NCHUNKS = _E // _CH
_CITER = 80
_STAGE = _CITER // 2
_NCHP = _NW * _CITER
_CNTP = 40960
_CSLICE = _CNTP // _NS
_ROWS_A = 624
_ROWS_LAST = _N - 15 * _ROWS_A

_mesh = plsc.VectorSubcoreMesh(core_axis_name="c", subcore_axis_name="s")


@functools.partial(
    pl.kernel,
    out_type=jax.ShapeDtypeStruct((_NC, _CNTP), jnp.float32),
    mesh=_mesh,
    compiler_params=pltpu.CompilerParams(needs_layout_passes=False),
    scratch_types=[
        pltpu.VMEM((_CITER, _CH), jnp.int32),
        pltpu.VMEM((_CITER, _CH), jnp.int32),
        pltpu.VMEM((_CITER, _CH), jnp.int32),
        pltpu.VMEM((_CH,), jnp.float32),
        pltpu.VMEM((_CSLICE,), jnp.float32),
        pltpu.SemaphoreType.DMA,
        pltpu.VMEM_SHARED((_CNTP,), jnp.float32),
    ],
)
def _sc_counts(dst_hbm, et_hbm, out_hbm, dst_a, et_a, seg_a, one_v, zb_v,
               sem, acc_sh):
    c = lax.axis_index("c")
    s = lax.axis_index("s")
    wid = c * _NS + s
    zeros16 = jnp.zeros((16,), jnp.float32)
    for k in range(_CSLICE // 16):
        zb_v[pl.ds(k * 16, 16)] = zeros16
    for k in range(_CH // 16):
        one_v[pl.ds(k * 16, 16)] = jnp.full((16,), 1.0, jnp.float32)
    cb = wid * _CITER
    pltpu.sync_copy(dst_hbm.at[pl.ds(cb, _CITER)], dst_a)
    pltpu.sync_copy(et_hbm.at[pl.ds(cb, _CITER)], et_a)
    pltpu.sync_copy(zb_v, acc_sh.at[pl.ds(s * _CSLICE, _CSLICE)])
    plsc.subcore_barrier()

    def body(j, carry):
        for k in range(_CH // 16):
            sl = pl.ds(k * 16, 16)
            r16 = jnp.right_shift(et_a[j, sl], 1)
            seg_a[j, sl] = r16 * _N + dst_a[j, sl]
        pltpu.async_copy(one_v, acc_sh.at[seg_a.at[j]], sem, add=True)
        return carry

    lax.fori_loop(0, _CITER, body, 0)

    def drain(j, carry):
        pltpu.make_async_copy(one_v, acc_sh.at[seg_a.at[j]], sem).wait()
        return carry

    lax.fori_loop(0, _CITER, drain, 0)
    plsc.subcore_barrier()
    pltpu.sync_copy(acc_sh.at[pl.ds(s * _CSLICE, _CSLICE)],
                    out_hbm.at[c, pl.ds(s * _CSLICE, _CSLICE)])


@functools.partial(
    pl.kernel,
    out_type=(jax.ShapeDtypeStruct((_NCHP, _CH), jnp.float32),
              jax.ShapeDtypeStruct((_NCHP, _CH), jnp.int32)),
    mesh=_mesh,
    compiler_params=pltpu.CompilerParams(needs_layout_passes=False),
    scratch_types=[
        pltpu.VMEM((_CITER, _CH), jnp.int32),
        pltpu.VMEM((_CITER, _CH), jnp.int32),
        pltpu.VMEM((_CITER, _CH), jnp.int32),
        pltpu.VMEM((_CITER, _CH), jnp.float32),
        pltpu.VMEM((_CITER, _CH), jnp.int32),
        pltpu.VMEM((_CNTP,), jnp.float32),
    ],
)
def _sc_edge_prep(src_hbm, dst_hbm, et_hbm, inv_hbm,
                  w_hbm, gidx_hbm, src_a, dst_a, et_a, w_a, g_a, inv_v):
    c = lax.axis_index("c")
    s = lax.axis_index("s")
    wid = c * _NS + s
    cb = wid * _CITER
    pltpu.sync_copy(inv_hbm, inv_v)
    pltpu.sync_copy(src_hbm.at[pl.ds(cb, _CITER)], src_a)
    pltpu.sync_copy(dst_hbm.at[pl.ds(cb, _CITER)], dst_a)
    pltpu.sync_copy(et_hbm.at[pl.ds(cb, _CITER)], et_a)

    def body(j, carry):
        for k in range(_CH // 16):
            sl = pl.ds(k * 16, 16)
            r16 = jnp.right_shift(et_a[j, sl], 1)
            seg16 = r16 * _N + dst_a[j, sl]
            w_a[j, sl] = plsc.load_gather(inv_v, [seg16])
            g_a[j, sl] = (r16 & 3) * _N + src_a[j, sl]
        return carry

    lax.fori_loop(0, _CITER, body, 0)
    pltpu.sync_copy(w_a, w_hbm.at[pl.ds(cb, _CITER)])
    pltpu.sync_copy(g_a, gidx_hbm.at[pl.ds(cb, _CITER)])


@functools.partial(
    pl.kernel,
    out_type=jax.ShapeDtypeStruct((_NC, _N, _D), jnp.float32),
    mesh=_mesh,
    compiler_params=pltpu.CompilerParams(needs_layout_passes=False),
    scratch_types=[
        pltpu.VMEM((_STAGE, _CH), jnp.int32),
        pltpu.VMEM((_STAGE, _CH), jnp.int32),
        pltpu.VMEM((_STAGE, _CH), jnp.float32),
        pltpu.VMEM((_CH, _D), jnp.float32),
        pltpu.VMEM((_CH, _D), jnp.float32),
        pltpu.SemaphoreType.DMA,
        pltpu.SemaphoreType.DMA,
        pltpu.VMEM_SHARED((_N, _D), jnp.float32),
    ],
)
def _sc_accumulate(z_hbm, gidx_hbm, dst_hbm, w_hbm, out_hbm,
                   gi_a, dst_a, w_a, rows0, rows1, sem0, sem1, acc_sh):
    c = lax.axis_index("c")
    s = lax.axis_index("s")
    wid = c * _NS + s
    zeros16 = jnp.zeros((16,), jnp.float32)

    def zrow(i, carry):
        for k in range(_D // 16):
            rows0[i, pl.ds(k * 16, 16)] = zeros16
        return carry

    lax.fori_loop(0, _CH, zrow, 0)
    base = s * _ROWS_A
    for q in range(4):
        pltpu.sync_copy(rows0, acc_sh.at[pl.ds(base + q * _CH, _CH)])

    @pl.when(s < _NS - 1)
    def _():
        pltpu.sync_copy(rows0.at[pl.ds(0, _ROWS_A - 4 * _CH)],
                        acc_sh.at[pl.ds(base + 4 * _CH, _ROWS_A - 4 * _CH)])

    @pl.when(s == _NS - 1)
    def _():
        pltpu.sync_copy(rows0, acc_sh.at[pl.ds(15 * _ROWS_A + 4 * _CH, _CH)])

    plsc.subcore_barrier()

    def scale(rows_v, j):
        jsp = jnp.full((16,), j, jnp.int32)

        def srow(i4, carry2):
            i = i4 * 4
            wsps = [
                plsc.load_gather(w_a, [jsp, jnp.full((16,), i + u, jnp.int32)])
                for u in range(4)
            ]
            for u in range(4):
                for k in range(_D // 16):
                    sl = pl.ds(k * 16, 16)
                    rows_v[i + u, sl] = rows_v[i + u, sl] * wsps[u]
            return carry2

        lax.fori_loop(0, _CH // 4, srow, 0)

    cb = wid * _CITER
    for half in range(2):
        sb = cb + half * _STAGE
        pltpu.sync_copy(gidx_hbm.at[pl.ds(sb, _STAGE)], gi_a)
        pltpu.sync_copy(dst_hbm.at[pl.ds(sb, _STAGE)], dst_a)
        pltpu.sync_copy(w_hbm.at[pl.ds(sb, _STAGE)], w_a)
        pltpu.async_copy(z_hbm.at[gi_a.at[0]], rows0, sem0)

        def body(i, carry):
            j0 = i * 2
            pltpu.async_copy(z_hbm.at[gi_a.at[j0 + 1]], rows1, sem1)
            pltpu.make_async_copy(z_hbm.at[gi_a.at[j0]], rows0, sem0).wait()
            scale(rows0, j0)
            pltpu.sync_copy(rows0, acc_sh.at[dst_a.at[j0]], add=True)

            @pl.when(j0 + 2 < _STAGE)
            def _():
                pltpu.async_copy(z_hbm.at[gi_a.at[j0 + 2]], rows0, sem0)

            pltpu.make_async_copy(z_hbm.at[gi_a.at[j0 + 1]], rows1, sem1).wait()
            scale(rows1, j0 + 1)
            pltpu.sync_copy(rows1, acc_sh.at[dst_a.at[j0 + 1]], add=True)
            return carry

        lax.fori_loop(0, _STAGE // 2, body, 0)
    plsc.subcore_barrier()

    @pl.when(s < _NS - 1)
    def _():
        pltpu.sync_copy(acc_sh.at[pl.ds(s * _ROWS_A, _ROWS_A)],
                        out_hbm.at[c, pl.ds(s * _ROWS_A, _ROWS_A)])

    @pl.when(s == _NS - 1)
    def _():
        pltpu.sync_copy(acc_sh.at[pl.ds(15 * _ROWS_A, _ROWS_LAST)],
                        out_hbm.at[c, pl.ds(15 * _ROWS_A, _ROWS_LAST)])


def _proj_body(x_ref, w_ref, b_ref, o_ref):
    o_ref[...] = jnp.dot(x_ref[...], w_ref[...],
                         preferred_element_type=jnp.float32) + b_ref[...]


def _tc_project(x, w, b, block):
    m, k = x.shape
    return pl.pallas_call(
        _proj_body,
        grid=(m // block,),
        in_specs=[
            pl.BlockSpec((block, k), lambda i: (i, 0)),
            pl.BlockSpec((k, _D), lambda i: (0, 0)),
            pl.BlockSpec((1, _D), lambda i: (0, 0)),
        ],
        out_specs=pl.BlockSpec((block, _D), lambda i: (i, 0)),
        out_shape=jax.ShapeDtypeStruct((m, _D), jnp.float32),
    )(x, w, b.reshape(1, _D))


def _inv_body(c_ref, o_ref):
    inv = 1.0 / jnp.maximum(c_ref[0] + c_ref[1], 1.0)
    rows = c_ref.shape[1]
    flat = (lax.broadcasted_iota(jnp.int32, (rows, _D), 0) * _D
            + lax.broadcasted_iota(jnp.int32, (rows, _D), 1))
    o_ref[...] = jnp.where(flat < _R * _N, inv, 0.0)


def _tc_inv_counts(cnt2):
    c3 = cnt2.reshape(_NC, _CNTP // _D, _D)
    out = pl.pallas_call(
        _inv_body,
        out_shape=jax.ShapeDtypeStruct((_CNTP // _D, _D), jnp.float32),
    )(c3)
    return out.reshape(_CNTP)


def _z_body(x_ref, comp_ref, bases_ref, o_ref):
    for r in range(_R):
        wt = comp_ref[r, 0] * bases_ref[0]
        for b in range(1, bases_ref.shape[0]):
            wt = wt + comp_ref[r, b] * bases_ref[b]
        o_ref[r] = jnp.dot(x_ref[...], wt, preferred_element_type=jnp.float32)


def _tc_ztable(x, comp, bases, block):
    nb = bases.shape[0]
    return pl.pallas_call(
        _z_body,
        grid=(_N // block,),
        in_specs=[
            pl.BlockSpec((block, _D), lambda i: (i, 0)),
            pl.BlockSpec(memory_space=pltpu.SMEM),
            pl.BlockSpec((nb, _D, _D), lambda i: (0, 0, 0)),
        ],
        out_specs=pl.BlockSpec((_R, block, _D), lambda i: (0, i, 0)),
        out_shape=jax.ShapeDtypeStruct((_R, _N, _D), jnp.float32),
    )(x, comp, bases)


def _combine_body(acc_ref, x_ref, root_ref, b_ref, g_ref, bb_ref, o_ref, *, act):
    h = (acc_ref[0] + acc_ref[1] + b_ref[...]
         + jnp.dot(x_ref[...], root_ref[...], preferred_element_type=jnp.float32))
    if act == "relu":
        h = jnp.maximum(h, 0.0)
    elif act == "ln":
        mu = jnp.mean(h, axis=-1, keepdims=True)
        xc = h - mu
        var = jnp.mean(xc * xc, axis=-1, keepdims=True)
        h = xc * lax.rsqrt(var + 1e-5) * g_ref[...] + bb_ref[...]
    o_ref[...] = h


def _tc_combine(acc2, x, root, bias, g, bb, act, block):
    body = functools.partial(_combine_body, act=act)
    return pl.pallas_call(
        body,
        grid=(_N // block,),
        in_specs=[
            pl.BlockSpec((_NC, block, _D), lambda i: (0, i, 0)),
            pl.BlockSpec((block, _D), lambda i: (i, 0)),
            pl.BlockSpec((_D, _D), lambda i: (0, 0)),
            pl.BlockSpec((1, _D), lambda i: (0, 0)),
            pl.BlockSpec((1, _D), lambda i: (0, 0)),
            pl.BlockSpec((1, _D), lambda i: (0, 0)),
        ],
        out_specs=pl.BlockSpec((block, _D), lambda i: (i, 0)),
        out_shape=jax.ShapeDtypeStruct((_N, _D), jnp.float32),
    )(acc2, x, root, bias.reshape(1, _D), g.reshape(1, _D), bb.reshape(1, _D))


def kernel(x0, x1, edge_index, edge_type, lin0_w, lin0_b, lin1_w, lin1_b,
           comp1, bases1, root1, bias1, comp2, bases2, root2, bias2,
           comp3, bases3, root3, bias3, ln_g, ln_b):
    block = 1000
    pad = 384 - x1.shape[1]
    x1p = jnp.pad(x1, ((0, 0), (0, pad)))
    w1p = jnp.pad(lin1_w, ((0, pad), (0, 0)))
    xa = _tc_project(x0, lin0_w, lin0_b, 1000)
    xb = _tc_project(x1p, w1p, lin1_b, 1000)
    x = jnp.concatenate([xa, xb], axis=0)

    npad = _NCHP * _CH - _E
    pad_idx = jnp.arange(npad, dtype=jnp.int32)
    srcp = jnp.concatenate([edge_index[0], pad_idx % _N]).reshape(_NCHP, _CH)
    dstp = jnp.concatenate([edge_index[1], pad_idx % 960]).reshape(_NCHP, _CH)
    etp = jnp.concatenate(
        [edge_type, jnp.full((npad,), 8, jnp.int32)]).reshape(_NCHP, _CH)
    cnt2 = _sc_counts(dstp, etp)
    inv = _tc_inv_counts(cnt2)
    w_e, gidx = _sc_edge_prep(srcp, dstp, etp, inv)

    layers = ((comp1, bases1, root1, bias1, "relu"),
              (comp2, bases2, root2, bias2, "relu"),
              (comp3, bases3, root3, bias3, "ln"))
    for comp, bases, root, bias, act in layers:
        z = _tc_ztable(x, comp, bases, block).reshape(_R * _N, _D)
        acc2 = _sc_accumulate(z, gidx, dstp, w_e)
        x = _tc_combine(acc2, x, root, bias, ln_g, ln_b, act, block)
    return x
